# baseline (speedup 1.0000x reference)
import functools

import jax
import jax.numpy as jnp
from jax import lax
from jax.experimental import pallas as pl
from jax.experimental.pallas import tpu as pltpu
from jax.experimental.pallas import tpu_sc as plsc

CTX = 200
PATH = 20
EMBED = 64
LANES = 16
NVREG = EMBED // LANES
CTX_ROWS = 2
PATH_PAD = 32

_mesh = plsc.VectorSubcoreMesh(core_axis_name="c", subcore_axis_name="s")


@functools.partial(
    pl.kernel,
    out_type=(
        jax.ShapeDtypeStruct((EMBED,), jnp.float32),
        jax.ShapeDtypeStruct((PATH_PAD, EMBED), jnp.float32),
    ),
    mesh=_mesh,
    compiler_params=pltpu.CompilerParams(use_tc_tiling_on_sc=False),
    scratch_types=[
        pltpu.VMEM((CTX_ROWS, 128), jnp.int32),
        pltpu.VMEM((PATH_PAD,), jnp.int32),
        pltpu.VMEM((CTX_ROWS, 128, EMBED), jnp.float32),
        pltpu.VMEM((PATH_PAD, EMBED), jnp.float32),
        pltpu.VMEM((EMBED,), jnp.float32),
        pltpu.SemaphoreType.DMA,
    ],
)
def _sc_gather_pool(ctx_idx_hbm, path_idx_hbm, ctx_table_hbm, node_table_hbm,
                    h_hbm, nrows_hbm, idx_v, pidx_v, crows_v, nrows_v, h_v,
                    sem):
    wid = lax.axis_index("s") * _mesh.num_cores + lax.axis_index("c")

    @pl.when(wid == 0)
    def _():
        pltpu.sync_copy(ctx_idx_hbm, idx_v)
        pltpu.sync_copy(path_idx_hbm, pidx_v)
        c0 = pltpu.async_copy(ctx_table_hbm.at[idx_v.at[0]], crows_v.at[0], sem)
        c1 = pltpu.async_copy(ctx_table_hbm.at[idx_v.at[1]], crows_v.at[1], sem)
        c2 = pltpu.async_copy(node_table_hbm.at[pidx_v], nrows_v, sem)
        c0.wait()
        c1.wait()
        c2.wait()

        def body0(i, acc):
            return tuple(acc[k] + crows_v[0, i, pl.ds(LANES * k, LANES)]
                         for k in range(NVREG))

        def body1(i, acc):
            return tuple(acc[k] + crows_v[1, i, pl.ds(LANES * k, LANES)]
                         for k in range(NVREG))

        zero = tuple(jnp.zeros((LANES,), jnp.float32) for _ in range(NVREG))
        acc = lax.fori_loop(0, 128, body0, zero)
        acc = lax.fori_loop(0, CTX - 128, body1, acc)
        for k in range(NVREG):
            h_v[pl.ds(LANES * k, LANES)] = acc[k] * (1.0 / CTX)

        pltpu.sync_copy(h_v, h_hbm)
        pltpu.sync_copy(nrows_v, nrows_hbm)


def _loss_body(h_ref, n_ref, bits_ref, o_ref):
    h = h_ref[...]
    n = n_ref[...]
    b = bits_ref[...]
    t = jnp.sum(h * n, axis=1)[None, :]
    lane = lax.broadcasted_iota(jnp.int32, (1, PATH_PAD), 1)
    s = jax.nn.sigmoid(t)
    eps = 1e-9
    per = -b * jnp.log(s + eps) - (1.0 - b) * jnp.log(1.0 - s + eps)
    per = jnp.where(lane < PATH, per, 0.0)
    o_ref[0, 0] = jnp.sum(per)


_loss_call = pl.pallas_call(
    _loss_body,
    out_shape=jax.ShapeDtypeStruct((1, 1), jnp.float32),
    out_specs=pl.BlockSpec(memory_space=pltpu.SMEM),
)


def kernel(context_idx, path_indices, code_bits, context_table, node_table):
    ctx = jnp.asarray(context_idx, jnp.int32)
    ctx_pad = (jnp.zeros((CTX_ROWS * 128,), jnp.int32)
               .at[:CTX].set(ctx).reshape(CTX_ROWS, 128))
    path_pad = (jnp.zeros((PATH_PAD,), jnp.int32)
                .at[:PATH].set(jnp.asarray(path_indices, jnp.int32)))
    h, nrows = _sc_gather_pool(ctx_pad, path_pad, context_table, node_table)
    bits = (jnp.zeros((PATH_PAD,), jnp.float32)
            .at[:PATH].set(code_bits.astype(jnp.float32)))
    out = _loss_call(h.reshape(1, EMBED), nrows, bits.reshape(1, PATH_PAD))
    return out[0, 0]
